# Initial kernel scaffold; baseline (speedup 1.0000x reference)
import jax
import jax.numpy as jnp
from jax.experimental import pallas as pl


def kernel(enc_input_ids, dec_input_ids, dec_embed):
    raise NotImplementedError("write your pallas kernel here")


# speedup vs baseline: 4.8843x; 4.8843x over previous
import functools

import jax
import jax.numpy as jnp
from jax import lax
from jax.experimental import pallas as pl
from jax.experimental.pallas import tpu as pltpu
from jax.experimental.pallas import tpu_sc as plsc

_B, _T, _D = 16384, 200, 4
_N = _B * _T
_NW = 32
_PER_W = _N // _NW
_C = 6400
_CHUNKS = _PER_W // _C


def _sc_embed(idx_flat, table_flat):
    mesh = plsc.VectorSubcoreMesh(core_axis_name="c", subcore_axis_name="s")

    @functools.partial(
        pl.kernel,
        mesh=mesh,
        out_type=jax.ShapeDtypeStruct((_N * _D,), jnp.float32),
        compiler_params=pltpu.CompilerParams(needs_layout_passes=False),
        scratch_types=[
            pltpu.VMEM((32,), jnp.float32),
            pltpu.VMEM((_C,), jnp.int32),
            pltpu.VMEM((_C * _D,), jnp.float32),
        ],
    )
    def k(idx_hbm, tab_hbm, out_hbm, tab_v, idx_v, out_v):
        wid = lax.axis_index("s") * 2 + lax.axis_index("c")
        pltpu.sync_copy(tab_hbm, tab_v)
        lane = lax.iota(jnp.int32, 16)
        pat = lax.shift_right_logical(lane, 2)
        kpat = lax.bitwise_and(lane, 3)

        def chunk_body(cidx, _):
            base = wid * _PER_W + cidx * _C
            pltpu.sync_copy(idx_hbm.at[pl.ds(base, _C)], idx_v)

            def vec_body(m, _):
                offs = jnp.broadcast_to(m * 4, (16,)) + pat
                idxg = plsc.load_gather(idx_v, [offs])
                fidx = lax.shift_left(idxg, 2) + kpat
                vals = plsc.load_gather(tab_v, [fidx])
                out_v[pl.ds(m * 16, 16)] = vals
                return 0

            lax.fori_loop(0, _C // 4, vec_body, 0)
            pltpu.sync_copy(out_v, out_hbm.at[pl.ds(base * _D, _C * _D)])
            return 0

        lax.fori_loop(0, _CHUNKS, chunk_body, 0)

    return k(idx_flat, table_flat)


def kernel(enc_input_ids, dec_input_ids, dec_embed):
    del enc_input_ids
    idx_flat = dec_input_ids.reshape(-1).astype(jnp.int32)
    table_flat = dec_embed.reshape(-1)
    out = _sc_embed(idx_flat, table_flat)
    return out.reshape(_B, _T, _D)


# speedup vs baseline: 43.4983x; 8.9057x over previous
import functools

import jax
import jax.numpy as jnp
from jax import lax
from jax.experimental import pallas as pl
from jax.experimental.pallas import tpu as pltpu
from jax.experimental.pallas import tpu_sc as plsc

_B, _T, _D = 16384, 200, 4
_NW = 32
_BT = _B // 128
_BT_W = _BT // _NW
_TSEG = 50
_NSEG = _T // _TSEG


def _sc_embed(idx, table):
    mesh = plsc.VectorSubcoreMesh(core_axis_name="c", subcore_axis_name="s")

    @functools.partial(
        pl.kernel,
        mesh=mesh,
        out_type=jax.ShapeDtypeStruct((_T, _BT, _D, 128), jnp.float32),
        compiler_params=pltpu.CompilerParams(
            needs_layout_passes=False,
            use_tc_tiling_on_sc=False,
        ),
        scratch_types=[
            pltpu.VMEM((8, 4), jnp.float32),
            pltpu.VMEM((128, _T), jnp.int32),
            pltpu.VMEM((_TSEG, 1, _D, 128), jnp.float32),
        ],
    )
    def k(idx_hbm, tab_hbm, out_hbm, tab_v, idx_v, out_v):
        wid = lax.axis_index("s") * 2 + lax.axis_index("c")
        pltpu.sync_copy(tab_hbm, tab_v)
        lane = lax.iota(jnp.int32, 16)

        def bt_body(i, _):
            bt = wid * _BT_W + i
            pltpu.sync_copy(idx_hbm.at[pl.ds(bt * 128, 128), :], idx_v)

            def seg_body(s, _):
                t0 = s * _TSEG

                def t_body(tt, _):
                    tcol = jnp.broadcast_to(t0 + tt, (16,))
                    for g in range(8):
                        rows = jnp.broadcast_to(g * 16, (16,)) + lane
                        idxg = plsc.load_gather(idx_v, [rows, tcol])
                        for kk in range(4):
                            vals = plsc.load_gather(
                                tab_v, [idxg, jnp.broadcast_to(kk, (16,))]
                            )
                            out_v[tt, 0, kk, pl.ds(g * 16, 16)] = vals
                    return 0

                lax.fori_loop(0, _TSEG, t_body, 0)
                pltpu.sync_copy(
                    out_v,
                    out_hbm.at[pl.ds(t0, _TSEG), pl.ds(bt, 1), :, :],
                )
                return 0

            lax.fori_loop(0, _NSEG, seg_body, 0)
            return 0

        lax.fori_loop(0, _BT_W, bt_body, 0)

    return k(idx, table)


def kernel(enc_input_ids, dec_input_ids, dec_embed):
    del enc_input_ids
    out_phys = _sc_embed(dec_input_ids, dec_embed)
    return out_phys.transpose(1, 3, 0, 2).reshape(_B, _T, _D)


# speedup vs baseline: 97.0352x; 2.2308x over previous
import functools

import jax
import jax.numpy as jnp
from jax import lax
from jax.experimental import pallas as pl
from jax.experimental.pallas import tpu as pltpu
from jax.experimental.pallas import tpu_sc as plsc

_B, _T, _D = 16384, 200, 4
_NW = 32
_BT = _B // 128
_BT_W = _BT // _NW
_TSEG = 50
_NSEG = _T // _TSEG


def _sc_embed(idx, table):
    mesh = plsc.VectorSubcoreMesh(core_axis_name="c", subcore_axis_name="s")

    @functools.partial(
        pl.kernel,
        mesh=mesh,
        out_type=jax.ShapeDtypeStruct((_T, _BT, _D, 128), jnp.float32),
        compiler_params=pltpu.CompilerParams(
            needs_layout_passes=False,
            use_tc_tiling_on_sc=False,
        ),
        scratch_types=[
            pltpu.VMEM((8, 4), jnp.float32),
            pltpu.VMEM((128, _T), jnp.int32),
            pltpu.VMEM((_TSEG, 1, _D, 128), jnp.float32),
        ],
    )
    def k(idx_hbm, tab_hbm, out_hbm, tab_v, idx_v, out_v):
        wid = lax.axis_index("s") * 2 + lax.axis_index("c")
        pltpu.sync_copy(tab_hbm, tab_v)
        lane = lax.iota(jnp.int32, 16)

        def bt_body(i, _):
            bt = wid * _BT_W + i
            pltpu.sync_copy(idx_hbm.at[pl.ds(bt * 128, 128), :], idx_v)

            def seg_body(s, _):
                t0 = s * _TSEG

                @plsc.parallel_loop(0, _TSEG, unroll=2)
                def t_body(tt):
                    tcol = jnp.broadcast_to(t0 + tt, (16,))
                    for g in range(8):
                        rows = jnp.broadcast_to(g * 16, (16,)) + lane
                        idxg = plsc.load_gather(idx_v, [rows, tcol])
                        for kk in range(4):
                            vals = plsc.load_gather(
                                tab_v, [idxg, jnp.broadcast_to(kk, (16,))]
                            )
                            out_v[tt, 0, kk, pl.ds(g * 16, 16)] = vals
                pltpu.sync_copy(
                    out_v,
                    out_hbm.at[pl.ds(t0, _TSEG), pl.ds(bt, 1), :, :],
                )
                return 0

            lax.fori_loop(0, _NSEG, seg_body, 0)
            return 0

        lax.fori_loop(0, _BT_W, bt_body, 0)

    return k(idx, table)


def kernel(enc_input_ids, dec_input_ids, dec_embed):
    del enc_input_ids
    out_phys = _sc_embed(dec_input_ids, dec_embed)
    return out_phys.transpose(1, 3, 0, 2).reshape(_B, _T, _D)
